# baseline (speedup 1.0000x reference)
import functools

import jax
import jax.numpy as jnp
from jax import lax
from jax.experimental import pallas as pl
from jax.experimental.pallas import tpu as pltpu
from jax.experimental.pallas import tpu_sc as plsc

_T, _D, _E, _DFF = 2048, 1024, 8, 2048
_BM = 256
_NBLK = 24
_NPAD = _NBLK * _BM
_NW = 32
_TPW = _T // _NW
_CHT = 32


def _router_top2(xt, gate):
    logits = lax.dot_general(xt, gate, (((1,), (1,)), ((), ())),
                             preferred_element_type=jnp.float32)
    mx = jnp.max(logits, axis=-1, keepdims=True)
    ex = jnp.exp(logits - mx)
    probs = ex / jnp.sum(ex, axis=-1, keepdims=True)
    eidx = lax.broadcasted_iota(jnp.int32, probs.shape, 1)
    m1 = jnp.max(probs, axis=-1, keepdims=True)
    e1 = jnp.min(jnp.where(probs == m1, eidx, _E), axis=-1, keepdims=True)
    probs2 = jnp.where(eidx == e1, -1.0, probs)
    m2 = jnp.max(probs2, axis=-1, keepdims=True)
    e2 = jnp.min(jnp.where(probs2 == m2, eidx, _E), axis=-1, keepdims=True)
    return m1, e1, m2, e2


def _gelu_exact(h):
    return 0.5 * h * (1.0 + lax.erf(h * 0.7071067811865476))


def _router_body(x_ref, gate_ref, pos0_ref, pos1_ref, w0_ref, w1_ref, be_ref):
    xt = x_ref[...]
    m1, e1, m2, e2 = _router_top2(xt, gate_ref[...])
    eidx = lax.broadcasted_iota(jnp.int32, (_T, _E), 1)
    oh0 = (eidx == e1).astype(jnp.float32)
    oh1 = (eidx == e2).astype(jnp.float32)

    TR = 512
    ti = lax.broadcasted_iota(jnp.int32, (TR, TR), 0)
    tj = lax.broadcasted_iota(jnp.int32, (TR, TR), 1)
    tri = (tj < ti).astype(jnp.float32)

    def chunked_excl_cumsum(oh):
        outs = []
        carry = jnp.zeros((1, _E), jnp.float32)
        for c in range(_T // TR):
            blk = oh[c * TR:(c + 1) * TR]
            outs.append(jnp.dot(tri, blk, preferred_element_type=jnp.float32)
                        + carry)
            carry = carry + jnp.sum(blk, axis=0, keepdims=True)
        return jnp.concatenate(outs, axis=0), carry

    ex0, c0 = chunked_excl_cumsum(oh0)
    ex1, c1 = chunked_excl_cumsum(oh1)
    rank0 = jnp.sum(oh0 * ex0, axis=1, keepdims=True)
    rank1 = jnp.sum(oh1 * (ex1 + c0), axis=1, keepdims=True)

    counts = c0 + c1
    nb = jnp.floor((counts + float(_BM - 1)) * (1.0 / _BM))
    i8 = lax.broadcasted_iota(jnp.int32, (_E, _E), 0)
    j8 = lax.broadcasted_iota(jnp.int32, (_E, _E), 1)
    tri8 = (i8 < j8).astype(jnp.float32)
    bo = jnp.dot(nb, tri8, preferred_element_type=jnp.float32)
    po = bo * float(_BM)

    pos0 = rank0 + jnp.sum(oh0 * po, axis=1, keepdims=True)
    pos1 = rank1 + jnp.sum(oh1 * po, axis=1, keepdims=True)
    pos0_ref[...] = pos0.astype(jnp.int32)
    pos1_ref[...] = pos1.astype(jnp.int32)
    w0_ref[...] = jnp.broadcast_to(m1, (_T, 16))
    w1_ref[...] = jnp.broadcast_to(m2, (_T, 16))

    jb = lax.broadcasted_iota(jnp.int32, (_NBLK, _E), 0).astype(jnp.float32)
    started = (jb >= jnp.broadcast_to(bo, (_NBLK, _E))).astype(jnp.float32)
    be = jnp.sum(started, axis=1, keepdims=True) - 1.0
    be_ref[...] = be.astype(jnp.int32)


def _router_call(x2, gate_w):
    return pl.pallas_call(
        _router_body,
        out_shape=[
            jax.ShapeDtypeStruct((_T, 1), jnp.int32),
            jax.ShapeDtypeStruct((_T, 1), jnp.int32),
            jax.ShapeDtypeStruct((_T, 16), jnp.float32),
            jax.ShapeDtypeStruct((_T, 16), jnp.float32),
            jax.ShapeDtypeStruct((_NBLK, 1), jnp.int32),
        ],
    )(x2, gate_w)


def _dispatch_sc(x2, pos0, pos1):
    mesh = plsc.VectorSubcoreMesh(core_axis_name="c", subcore_axis_name="s")

    @functools.partial(
        pl.kernel, mesh=mesh,
        out_type=jax.ShapeDtypeStruct((_NPAD, _D), jnp.float32),
        scratch_types=[
            pltpu.VMEM((_TPW,), jnp.int32),
            pltpu.VMEM((_TPW,), jnp.int32),
            pltpu.VMEM((_TPW, _D), jnp.float32),
            pltpu.SemaphoreType.DMA,
        ],
    )
    def dispatch(x_hbm, pos0_hbm, pos1_hbm, xs_hbm, idx0_v, idx1_v, rows_v,
                 sem):
        wid = lax.axis_index("s") * 2 + lax.axis_index("c")
        base = wid * _TPW
        pltpu.sync_copy(x_hbm.at[pl.ds(base, _TPW)], rows_v)
        pltpu.sync_copy(pos0_hbm.at[pl.ds(base, _TPW)], idx0_v)
        pltpu.sync_copy(pos1_hbm.at[pl.ds(base, _TPW)], idx1_v)
        c0 = pltpu.async_copy(rows_v, xs_hbm.at[idx0_v], sem)
        c1 = pltpu.async_copy(rows_v, xs_hbm.at[idx1_v], sem)
        c0.wait()
        c1.wait()

    return dispatch(x2, pos0, pos1)


def _experts_outer(be_ref, xs_hbm, w1_hbm, b1_hbm, w2_hbm, b2_hbm, ys_hbm):
    def inner(xs_blk, w1_blk, b1_blk, w2_blk, b2_blk, ys_blk):
        h = jnp.dot(xs_blk[...], w1_blk[0],
                    preferred_element_type=jnp.float32) + b1_blk[0]
        g = _gelu_exact(h)
        ys_blk[...] = lax.dot_general(
            g, w2_blk[0], (((1,), (0,)), ((), ())),
            preferred_element_type=jnp.float32) + b2_blk[0]

    lookahead = pl.Buffered(buffer_count=3, use_lookahead=True)
    pipeline = pltpu.emit_pipeline(
        inner,
        grid=(_NBLK,),
        in_specs=[
            pl.BlockSpec((_BM, _D), lambda j: (j, 0)),
            pl.BlockSpec((1, _D, _DFF), lambda j: (be_ref[j], 0, 0),
                         pipeline_mode=lookahead),
            pl.BlockSpec((1, 1, _DFF), lambda j: (be_ref[j], 0, 0)),
            pl.BlockSpec((1, _DFF, _D), lambda j: (be_ref[j], 0, 0),
                         pipeline_mode=lookahead),
            pl.BlockSpec((1, 1, _D), lambda j: (be_ref[j], 0, 0)),
        ],
        out_specs=[pl.BlockSpec((_BM, _D), lambda j: (j, 0))],
    )
    pipeline(xs_hbm, w1_hbm, b1_hbm, w2_hbm, b2_hbm, ys_hbm)


def _experts_call(be, xs, W1, b1, W2, b2):
    return pl.pallas_call(
        _experts_outer,
        in_specs=[
            pl.BlockSpec(memory_space=pltpu.SMEM),
            pl.BlockSpec(memory_space=pl.ANY),
            pl.BlockSpec(memory_space=pl.ANY),
            pl.BlockSpec(memory_space=pl.ANY),
            pl.BlockSpec(memory_space=pl.ANY),
            pl.BlockSpec(memory_space=pl.ANY),
        ],
        out_specs=pl.BlockSpec(memory_space=pl.ANY),
        out_shape=jax.ShapeDtypeStruct((_NPAD, _D), jnp.float32),
    )(be, xs, W1, b1.reshape(_E, 1, _DFF), W2, b2.reshape(_E, 1, _D))


def _combine_sc(ys, pos0, pos1, w0r, w1r):
    mesh = plsc.VectorSubcoreMesh(core_axis_name="c", subcore_axis_name="s")

    @functools.partial(
        pl.kernel, mesh=mesh,
        out_type=jax.ShapeDtypeStruct((_T, _D), jnp.float32),
        scratch_types=[
            pltpu.VMEM((_CHT,), jnp.int32),
            pltpu.VMEM((_CHT,), jnp.int32),
            pltpu.VMEM((_CHT, _D), jnp.float32),
            pltpu.VMEM((_CHT, _D), jnp.float32),
            pltpu.VMEM((_TPW, 16), jnp.float32),
            pltpu.VMEM((_TPW, 16), jnp.float32),
            pltpu.SemaphoreType.DMA,
        ],
    )
    def combine(ys_hbm, pos0_hbm, pos1_hbm, w0_hbm, w1_hbm, y_hbm,
                idx0_v, idx1_v, r0_v, r1_v, w0_v, w1_v, sem):
        wid = lax.axis_index("s") * 2 + lax.axis_index("c")
        base = wid * _TPW
        pltpu.sync_copy(w0_hbm.at[pl.ds(base, _TPW)], w0_v)
        pltpu.sync_copy(w1_hbm.at[pl.ds(base, _TPW)], w1_v)
        for ch in range(_TPW // _CHT):
            cbase = base + ch * _CHT
            pltpu.sync_copy(pos0_hbm.at[pl.ds(cbase, _CHT)], idx0_v)
            pltpu.sync_copy(pos1_hbm.at[pl.ds(cbase, _CHT)], idx1_v)
            g0 = pltpu.async_copy(ys_hbm.at[idx0_v], r0_v, sem)
            g1 = pltpu.async_copy(ys_hbm.at[idx1_v], r1_v, sem)
            g0.wait()
            g1.wait()

            def tok_body(t, _, ch=ch):
                wa = w0_v[ch * _CHT + t, :]
                wb = w1_v[ch * _CHT + t, :]
                for jc in range(_D // 16):
                    a = r0_v[t, pl.ds(jc * 16, 16)]
                    b = r1_v[t, pl.ds(jc * 16, 16)]
                    r0_v[t, pl.ds(jc * 16, 16)] = a * wa + b * wb
                return 0

            lax.fori_loop(0, _CHT, tok_body, 0)
            pltpu.sync_copy(r0_v, y_hbm.at[pl.ds(cbase, _CHT)])

    return combine(ys, pos0, pos1, w0r, w1r)


def kernel(x, gate_w, W1, b1, W2, b2):
    B_, T_, D_ = x.shape
    x2 = x.reshape(T_, D_)
    pos0, pos1, w0r, w1r, be = _router_call(x2, gate_w)
    pos0f = pos0.reshape(T_)
    pos1f = pos1.reshape(T_)
    xs = _dispatch_sc(x2, pos0f, pos1f)
    ys = _experts_call(be.reshape(_NBLK), xs, W1, b1, W2, b2)
    y = _combine_sc(ys, pos0f, pos1f, w0r, w1r)
    return y.reshape(B_, T_, D_)


# speedup vs baseline: 1.0040x; 1.0040x over previous
import functools

import jax
import jax.numpy as jnp
from jax import lax
from jax.experimental import pallas as pl
from jax.experimental.pallas import tpu as pltpu
from jax.experimental.pallas import tpu_sc as plsc

_T, _D, _E, _DFF = 2048, 1024, 8, 2048
_BM = 256
_NBLK = 24
_NPAD = _NBLK * _BM
_NW = 32
_TPW = _T // _NW
_CHT = 16


def _router_top2(xt, gate):
    logits = lax.dot_general(xt, gate, (((1,), (1,)), ((), ())),
                             preferred_element_type=jnp.float32)
    mx = jnp.max(logits, axis=-1, keepdims=True)
    ex = jnp.exp(logits - mx)
    probs = ex / jnp.sum(ex, axis=-1, keepdims=True)
    eidx = lax.broadcasted_iota(jnp.int32, probs.shape, 1)
    m1 = jnp.max(probs, axis=-1, keepdims=True)
    e1 = jnp.min(jnp.where(probs == m1, eidx, _E), axis=-1, keepdims=True)
    probs2 = jnp.where(eidx == e1, -1.0, probs)
    m2 = jnp.max(probs2, axis=-1, keepdims=True)
    e2 = jnp.min(jnp.where(probs2 == m2, eidx, _E), axis=-1, keepdims=True)
    return m1, e1, m2, e2


def _gelu_exact(h):
    return 0.5 * h * (1.0 + lax.erf(h * 0.7071067811865476))


def _router_body(x_ref, gate_ref, pos0_ref, pos1_ref, w0_ref, w1_ref, be_ref):
    xt = x_ref[...]
    m1, e1, m2, e2 = _router_top2(xt, gate_ref[...])
    eidx = lax.broadcasted_iota(jnp.int32, (_T, _E), 1)
    oh0 = (eidx == e1).astype(jnp.float32)
    oh1 = (eidx == e2).astype(jnp.float32)

    TR = 512
    ti = lax.broadcasted_iota(jnp.int32, (TR, TR), 0)
    tj = lax.broadcasted_iota(jnp.int32, (TR, TR), 1)
    tri = (tj < ti).astype(jnp.float32)

    def chunked_excl_cumsum(oh):
        outs = []
        carry = jnp.zeros((1, _E), jnp.float32)
        for c in range(_T // TR):
            blk = oh[c * TR:(c + 1) * TR]
            outs.append(jnp.dot(tri, blk, preferred_element_type=jnp.float32)
                        + carry)
            carry = carry + jnp.sum(blk, axis=0, keepdims=True)
        return jnp.concatenate(outs, axis=0), carry

    ex0, c0 = chunked_excl_cumsum(oh0)
    ex1, c1 = chunked_excl_cumsum(oh1)
    rank0 = jnp.sum(oh0 * ex0, axis=1, keepdims=True)
    rank1 = jnp.sum(oh1 * (ex1 + c0), axis=1, keepdims=True)

    counts = c0 + c1
    nb = jnp.floor((counts + float(_BM - 1)) * (1.0 / _BM))
    i8 = lax.broadcasted_iota(jnp.int32, (_E, _E), 0)
    j8 = lax.broadcasted_iota(jnp.int32, (_E, _E), 1)
    tri8 = (i8 < j8).astype(jnp.float32)
    bo = jnp.dot(nb, tri8, preferred_element_type=jnp.float32)
    po = bo * float(_BM)

    pos0 = rank0 + jnp.sum(oh0 * po, axis=1, keepdims=True)
    pos1 = rank1 + jnp.sum(oh1 * po, axis=1, keepdims=True)
    pos0_ref[...] = pos0.astype(jnp.int32)
    pos1_ref[...] = pos1.astype(jnp.int32)
    w0_ref[...] = jnp.broadcast_to(m1, (_T, 16))
    w1_ref[...] = jnp.broadcast_to(m2, (_T, 16))

    jb = lax.broadcasted_iota(jnp.int32, (_NBLK, _E), 0).astype(jnp.float32)
    started = (jb >= jnp.broadcast_to(bo, (_NBLK, _E))).astype(jnp.float32)
    be = jnp.sum(started, axis=1, keepdims=True) - 1.0
    be_ref[...] = be.astype(jnp.int32)


def _router_call(x2, gate_w):
    return pl.pallas_call(
        _router_body,
        out_shape=[
            jax.ShapeDtypeStruct((_T, 1), jnp.int32),
            jax.ShapeDtypeStruct((_T, 1), jnp.int32),
            jax.ShapeDtypeStruct((_T, 16), jnp.float32),
            jax.ShapeDtypeStruct((_T, 16), jnp.float32),
            jax.ShapeDtypeStruct((_NBLK, 1), jnp.int32),
        ],
    )(x2, gate_w)


def _dispatch_sc(x2, pos0, pos1):
    mesh = plsc.VectorSubcoreMesh(core_axis_name="c", subcore_axis_name="s")

    @functools.partial(
        pl.kernel, mesh=mesh,
        out_type=jax.ShapeDtypeStruct((_NPAD, _D), jnp.float32),
        scratch_types=[
            pltpu.VMEM((_TPW,), jnp.int32),
            pltpu.VMEM((_TPW,), jnp.int32),
            pltpu.VMEM((_TPW, _D), jnp.float32),
            pltpu.SemaphoreType.DMA,
        ],
    )
    def dispatch(x_hbm, pos0_hbm, pos1_hbm, xs_hbm, idx0_v, idx1_v, rows_v,
                 sem):
        wid = lax.axis_index("s") * 2 + lax.axis_index("c")
        base = wid * _TPW
        pltpu.sync_copy(x_hbm.at[pl.ds(base, _TPW)], rows_v)
        pltpu.sync_copy(pos0_hbm.at[pl.ds(base, _TPW)], idx0_v)
        pltpu.sync_copy(pos1_hbm.at[pl.ds(base, _TPW)], idx1_v)
        c0 = pltpu.async_copy(rows_v, xs_hbm.at[idx0_v], sem)
        c1 = pltpu.async_copy(rows_v, xs_hbm.at[idx1_v], sem)
        c0.wait()
        c1.wait()

    return dispatch(x2, pos0, pos1)


def _experts_outer(be_ref, xs_hbm, w1_hbm, b1_hbm, w2_hbm, b2_hbm, ys_hbm):
    def inner(xs_blk, w1_blk, b1_blk, w2_blk, b2_blk, ys_blk):
        h = jnp.dot(xs_blk[...], w1_blk[0],
                    preferred_element_type=jnp.float32) + b1_blk[0]
        g = _gelu_exact(h)
        ys_blk[...] = lax.dot_general(
            g, w2_blk[0], (((1,), (0,)), ((), ())),
            preferred_element_type=jnp.float32) + b2_blk[0]

    lookahead = pl.Buffered(buffer_count=2, use_lookahead=True)
    pipeline = pltpu.emit_pipeline(
        inner,
        grid=(_NBLK,),
        in_specs=[
            pl.BlockSpec((_BM, _D), lambda j: (j, 0)),
            pl.BlockSpec((1, _D, _DFF), lambda j: (be_ref[j], 0, 0),
                         pipeline_mode=lookahead),
            pl.BlockSpec((1, 1, _DFF), lambda j: (be_ref[j], 0, 0)),
            pl.BlockSpec((1, _DFF, _D), lambda j: (be_ref[j], 0, 0),
                         pipeline_mode=lookahead),
            pl.BlockSpec((1, 1, _D), lambda j: (be_ref[j], 0, 0)),
        ],
        out_specs=[pl.BlockSpec((_BM, _D), lambda j: (j, 0))],
    )
    pipeline(xs_hbm, w1_hbm, b1_hbm, w2_hbm, b2_hbm, ys_hbm)


def _experts_call(be, xs, W1, b1, W2, b2):
    return pl.pallas_call(
        _experts_outer,
        in_specs=[
            pl.BlockSpec(memory_space=pltpu.SMEM),
            pl.BlockSpec(memory_space=pl.ANY),
            pl.BlockSpec(memory_space=pl.ANY),
            pl.BlockSpec(memory_space=pl.ANY),
            pl.BlockSpec(memory_space=pl.ANY),
            pl.BlockSpec(memory_space=pl.ANY),
        ],
        out_specs=pl.BlockSpec(memory_space=pl.ANY),
        out_shape=jax.ShapeDtypeStruct((_NPAD, _D), jnp.float32),
    )(be, xs, W1, b1.reshape(_E, 1, _DFF), W2, b2.reshape(_E, 1, _D))


def _combine_sc(ys, pos0, pos1, w0r, w1r):
    mesh = plsc.VectorSubcoreMesh(core_axis_name="c", subcore_axis_name="s")

    ncha = _TPW // _CHT

    @functools.partial(
        pl.kernel, mesh=mesh,
        out_type=jax.ShapeDtypeStruct((_T, _D), jnp.float32),
        scratch_types=[
            pltpu.VMEM((2, _CHT), jnp.int32),
            pltpu.VMEM((2, _CHT), jnp.int32),
            pltpu.VMEM((2, _CHT, _D), jnp.float32),
            pltpu.VMEM((2, _CHT, _D), jnp.float32),
            pltpu.VMEM((_TPW, 16), jnp.float32),
            pltpu.VMEM((_TPW, 16), jnp.float32),
            pltpu.SemaphoreType.DMA,
            pltpu.SemaphoreType.DMA,
        ],
    )
    def combine(ys_hbm, pos0_hbm, pos1_hbm, w0_hbm, w1_hbm, y_hbm,
                idx0_v, idx1_v, r0_v, r1_v, w0_v, w1_v, semA, semB):
        wid = lax.axis_index("s") * 2 + lax.axis_index("c")
        base = wid * _TPW
        pltpu.sync_copy(w0_hbm.at[pl.ds(base, _TPW)], w0_v)
        pltpu.sync_copy(w1_hbm.at[pl.ds(base, _TPW)], w1_v)
        sems = [semA, semB]

        def start_chunk(c):
            b = c % 2
            cb = base + c * _CHT
            pltpu.sync_copy(pos0_hbm.at[pl.ds(cb, _CHT)], idx0_v.at[b])
            pltpu.sync_copy(pos1_hbm.at[pl.ds(cb, _CHT)], idx1_v.at[b])
            g0 = pltpu.async_copy(ys_hbm.at[idx0_v.at[b]], r0_v.at[b],
                                  sems[b])
            g1 = pltpu.async_copy(ys_hbm.at[idx1_v.at[b]], r1_v.at[b],
                                  sems[b])
            return g0, g1

        pend = start_chunk(0)
        for c in range(ncha):
            b = c % 2
            g0, g1 = pend
            if c + 1 < ncha:
                nxt = start_chunk(c + 1)
            g0.wait()
            g1.wait()

            def tok_body(t, _, c=c, b=b):
                wa = w0_v[c * _CHT + t, :]
                wb = w1_v[c * _CHT + t, :]
                for jc in range(_D // 16):
                    a = r0_v[b, t, pl.ds(jc * 16, 16)]
                    bb = r1_v[b, t, pl.ds(jc * 16, 16)]
                    r0_v[b, t, pl.ds(jc * 16, 16)] = a * wa + bb * wb
                return 0

            lax.fori_loop(0, _CHT, tok_body, 0)
            pltpu.sync_copy(r0_v.at[b], y_hbm.at[pl.ds(base + c * _CHT,
                                                       _CHT)])
            if c + 1 < ncha:
                pend = nxt

    return combine(ys, pos0, pos1, w0r, w1r)


def kernel(x, gate_w, W1, b1, W2, b2):
    B_, T_, D_ = x.shape
    x2 = x.reshape(T_, D_)
    pos0, pos1, w0r, w1r, be = _router_call(x2, gate_w)
    pos0f = pos0.reshape(T_)
    pos1f = pos1.reshape(T_)
    xs = _dispatch_sc(x2, pos0f, pos1f)
    ys = _experts_call(be.reshape(_NBLK), xs, W1, b1, W2, b2)
    y = _combine_sc(ys, pos0f, pos1f, w0r, w1r)
    return y.reshape(B_, T_, D_)


# speedup vs baseline: 1.0113x; 1.0073x over previous
import functools

import jax
import jax.numpy as jnp
from jax import lax
from jax.experimental import pallas as pl
from jax.experimental.pallas import tpu as pltpu
from jax.experimental.pallas import tpu_sc as plsc

_T, _D, _E, _DFF = 2048, 1024, 8, 2048
_BM = 256
_NBLK = 24
_NPAD = _NBLK * _BM
_NW = 32
_TPW = _T // _NW
_CHT = 32


def _router_top2(xt, gate):
    logits = lax.dot_general(xt, gate, (((1,), (1,)), ((), ())),
                             preferred_element_type=jnp.float32)
    mx = jnp.max(logits, axis=-1, keepdims=True)
    ex = jnp.exp(logits - mx)
    probs = ex / jnp.sum(ex, axis=-1, keepdims=True)
    eidx = lax.broadcasted_iota(jnp.int32, probs.shape, 1)
    m1 = jnp.max(probs, axis=-1, keepdims=True)
    e1 = jnp.min(jnp.where(probs == m1, eidx, _E), axis=-1, keepdims=True)
    probs2 = jnp.where(eidx == e1, -1.0, probs)
    m2 = jnp.max(probs2, axis=-1, keepdims=True)
    e2 = jnp.min(jnp.where(probs2 == m2, eidx, _E), axis=-1, keepdims=True)
    return m1, e1, m2, e2


def _gelu_exact(h):
    return 0.5 * h * (1.0 + lax.erf(h * 0.7071067811865476))


def _router_body(x_ref, gate_ref, pos0_ref, pos1_ref, w0_ref, w1_ref, be_ref):
    xt = x_ref[...]
    m1, e1, m2, e2 = _router_top2(xt, gate_ref[...])
    eidx = lax.broadcasted_iota(jnp.int32, (_T, _E), 1)
    oh0 = (eidx == e1).astype(jnp.float32)
    oh1 = (eidx == e2).astype(jnp.float32)

    TR = 512
    ti = lax.broadcasted_iota(jnp.int32, (TR, TR), 0)
    tj = lax.broadcasted_iota(jnp.int32, (TR, TR), 1)
    tri = (tj < ti).astype(jnp.float32)

    def chunked_excl_cumsum(oh):
        outs = []
        carry = jnp.zeros((1, _E), jnp.float32)
        for c in range(_T // TR):
            blk = oh[c * TR:(c + 1) * TR]
            outs.append(jnp.dot(tri, blk, preferred_element_type=jnp.float32)
                        + carry)
            carry = carry + jnp.sum(blk, axis=0, keepdims=True)
        return jnp.concatenate(outs, axis=0), carry

    ex0, c0 = chunked_excl_cumsum(oh0)
    ex1, c1 = chunked_excl_cumsum(oh1)
    rank0 = jnp.sum(oh0 * ex0, axis=1, keepdims=True)
    rank1 = jnp.sum(oh1 * (ex1 + c0), axis=1, keepdims=True)

    counts = c0 + c1
    nb = jnp.floor((counts + float(_BM - 1)) * (1.0 / _BM))
    i8 = lax.broadcasted_iota(jnp.int32, (_E, _E), 0)
    j8 = lax.broadcasted_iota(jnp.int32, (_E, _E), 1)
    tri8 = (i8 < j8).astype(jnp.float32)
    bo = jnp.dot(nb, tri8, preferred_element_type=jnp.float32)
    po = bo * float(_BM)

    pos0 = rank0 + jnp.sum(oh0 * po, axis=1, keepdims=True)
    pos1 = rank1 + jnp.sum(oh1 * po, axis=1, keepdims=True)
    pos0_ref[...] = pos0.astype(jnp.int32)
    pos1_ref[...] = pos1.astype(jnp.int32)
    w0_ref[...] = jnp.broadcast_to(m1, (_T, 16))
    w1_ref[...] = jnp.broadcast_to(m2, (_T, 16))

    jb = lax.broadcasted_iota(jnp.int32, (_NBLK, _E), 0).astype(jnp.float32)
    started = (jb >= jnp.broadcast_to(bo, (_NBLK, _E))).astype(jnp.float32)
    be = jnp.sum(started, axis=1, keepdims=True) - 1.0
    be_ref[...] = be.astype(jnp.int32)


def _router_call(x2, gate_w):
    return pl.pallas_call(
        _router_body,
        out_shape=[
            jax.ShapeDtypeStruct((_T, 1), jnp.int32),
            jax.ShapeDtypeStruct((_T, 1), jnp.int32),
            jax.ShapeDtypeStruct((_T, 16), jnp.float32),
            jax.ShapeDtypeStruct((_T, 16), jnp.float32),
            jax.ShapeDtypeStruct((_NBLK, 1), jnp.int32),
        ],
    )(x2, gate_w)


def _dispatch_sc(x2, pos0, pos1):
    mesh = plsc.VectorSubcoreMesh(core_axis_name="c", subcore_axis_name="s")

    @functools.partial(
        pl.kernel, mesh=mesh,
        out_type=jax.ShapeDtypeStruct((_NPAD, _D), jnp.float32),
        scratch_types=[
            pltpu.VMEM((_TPW,), jnp.int32),
            pltpu.VMEM((_TPW,), jnp.int32),
            pltpu.VMEM((_TPW, _D), jnp.float32),
            pltpu.SemaphoreType.DMA,
        ],
    )
    def dispatch(x_hbm, pos0_hbm, pos1_hbm, xs_hbm, idx0_v, idx1_v, rows_v,
                 sem):
        wid = lax.axis_index("s") * 2 + lax.axis_index("c")
        base = wid * _TPW
        pltpu.sync_copy(x_hbm.at[pl.ds(base, _TPW)], rows_v)
        pltpu.sync_copy(pos0_hbm.at[pl.ds(base, _TPW)], idx0_v)
        pltpu.sync_copy(pos1_hbm.at[pl.ds(base, _TPW)], idx1_v)
        c0 = pltpu.async_copy(rows_v, xs_hbm.at[idx0_v], sem)
        c1 = pltpu.async_copy(rows_v, xs_hbm.at[idx1_v], sem)
        c0.wait()
        c1.wait()

    return dispatch(x2, pos0, pos1)


def _experts_outer(be_ref, xs_hbm, w1_hbm, b1_hbm, w2_hbm, b2_hbm, ys_hbm):
    def inner(xs_blk, w1_blk, b1_blk, w2_blk, b2_blk, ys_blk):
        h = jnp.dot(xs_blk[...], w1_blk[0],
                    preferred_element_type=jnp.float32) + b1_blk[0]
        g = _gelu_exact(h)
        ys_blk[...] = lax.dot_general(
            g, w2_blk[0], (((1,), (0,)), ((), ())),
            preferred_element_type=jnp.float32) + b2_blk[0]

    lookahead = pl.Buffered(buffer_count=2, use_lookahead=True)
    pipeline = pltpu.emit_pipeline(
        inner,
        grid=(_NBLK,),
        in_specs=[
            pl.BlockSpec((_BM, _D), lambda j: (j, 0)),
            pl.BlockSpec((1, _D, _DFF), lambda j: (be_ref[j], 0, 0),
                         pipeline_mode=lookahead),
            pl.BlockSpec((1, 1, _DFF), lambda j: (be_ref[j], 0, 0)),
            pl.BlockSpec((1, _DFF, _D), lambda j: (be_ref[j], 0, 0),
                         pipeline_mode=lookahead),
            pl.BlockSpec((1, 1, _D), lambda j: (be_ref[j], 0, 0)),
        ],
        out_specs=[pl.BlockSpec((_BM, _D), lambda j: (j, 0))],
    )
    pipeline(xs_hbm, w1_hbm, b1_hbm, w2_hbm, b2_hbm, ys_hbm)


def _experts_call(be, xs, W1, b1, W2, b2):
    return pl.pallas_call(
        _experts_outer,
        in_specs=[
            pl.BlockSpec(memory_space=pltpu.SMEM),
            pl.BlockSpec(memory_space=pl.ANY),
            pl.BlockSpec(memory_space=pl.ANY),
            pl.BlockSpec(memory_space=pl.ANY),
            pl.BlockSpec(memory_space=pl.ANY),
            pl.BlockSpec(memory_space=pl.ANY),
        ],
        out_specs=pl.BlockSpec(memory_space=pl.ANY),
        out_shape=jax.ShapeDtypeStruct((_NPAD, _D), jnp.float32),
    )(be, xs, W1, b1.reshape(_E, 1, _DFF), W2, b2.reshape(_E, 1, _D))


def _combine_sc(ys, pos0, pos1, w0r, w1r):
    mesh = plsc.VectorSubcoreMesh(core_axis_name="c", subcore_axis_name="s")

    @functools.partial(
        pl.kernel, mesh=mesh,
        out_type=jax.ShapeDtypeStruct((_T, _D), jnp.float32),
        scratch_types=[
            pltpu.VMEM((_CHT,), jnp.int32),
            pltpu.VMEM((_CHT,), jnp.int32),
            pltpu.VMEM((_CHT, _D), jnp.float32),
            pltpu.VMEM((_CHT, _D), jnp.float32),
            pltpu.VMEM((_TPW, 16), jnp.float32),
            pltpu.VMEM((_TPW, 16), jnp.float32),
            pltpu.SemaphoreType.DMA,
        ],
    )
    def combine(ys_hbm, pos0_hbm, pos1_hbm, w0_hbm, w1_hbm, y_hbm,
                idx0_v, idx1_v, r0_v, r1_v, w0_v, w1_v, sem):
        wid = lax.axis_index("s") * 2 + lax.axis_index("c")
        base = wid * _TPW
        pltpu.sync_copy(w0_hbm.at[pl.ds(base, _TPW)], w0_v)
        pltpu.sync_copy(w1_hbm.at[pl.ds(base, _TPW)], w1_v)
        for ch in range(_TPW // _CHT):
            cbase = base + ch * _CHT
            pltpu.sync_copy(pos0_hbm.at[pl.ds(cbase, _CHT)], idx0_v)
            pltpu.sync_copy(pos1_hbm.at[pl.ds(cbase, _CHT)], idx1_v)
            g0 = pltpu.async_copy(ys_hbm.at[idx0_v], r0_v, sem)
            g1 = pltpu.async_copy(ys_hbm.at[idx1_v], r1_v, sem)
            g0.wait()
            g1.wait()

            def tok_body(t, _, ch=ch):
                wa = w0_v[ch * _CHT + t, :]
                wb = w1_v[ch * _CHT + t, :]
                for jc in range(_D // 16):
                    a = r0_v[t, pl.ds(jc * 16, 16)]
                    b = r1_v[t, pl.ds(jc * 16, 16)]
                    r0_v[t, pl.ds(jc * 16, 16)] = a * wa + b * wb
                return 0

            lax.fori_loop(0, _CHT, tok_body, 0)
            pltpu.sync_copy(r0_v, y_hbm.at[pl.ds(cbase, _CHT)])

    return combine(ys, pos0, pos1, w0r, w1r)


def kernel(x, gate_w, W1, b1, W2, b2):
    B_, T_, D_ = x.shape
    x2 = x.reshape(T_, D_)
    pos0, pos1, w0r, w1r, be = _router_call(x2, gate_w)
    pos0f = pos0.reshape(T_)
    pos1f = pos1.reshape(T_)
    xs = _dispatch_sc(x2, pos0f, pos1f)
    ys = _experts_call(be.reshape(_NBLK), xs, W1, b1, W2, b2)
    y = _combine_sc(ys, pos0f, pos1f, w0r, w1r)
    return y.reshape(B_, T_, D_)


# speedup vs baseline: 1.0378x; 1.0262x over previous
import functools

import jax
import jax.numpy as jnp
from jax import lax
from jax.experimental import pallas as pl
from jax.experimental.pallas import tpu as pltpu
from jax.experimental.pallas import tpu_sc as plsc

_T, _D, _E, _DFF = 2048, 1024, 8, 2048
_BM = 256
_NBLK = 24
_NPAD = _NBLK * _BM
_NW = 32
_TPW = _T // _NW
_CHT = 32


def _router_top2(xt, gate):
    logits = lax.dot_general(xt, gate, (((1,), (1,)), ((), ())),
                             preferred_element_type=jnp.float32)
    mx = jnp.max(logits, axis=-1, keepdims=True)
    ex = jnp.exp(logits - mx)
    probs = ex / jnp.sum(ex, axis=-1, keepdims=True)
    eidx = lax.broadcasted_iota(jnp.int32, probs.shape, 1)
    m1 = jnp.max(probs, axis=-1, keepdims=True)
    e1 = jnp.min(jnp.where(probs == m1, eidx, _E), axis=-1, keepdims=True)
    probs2 = jnp.where(eidx == e1, -1.0, probs)
    m2 = jnp.max(probs2, axis=-1, keepdims=True)
    e2 = jnp.min(jnp.where(probs2 == m2, eidx, _E), axis=-1, keepdims=True)
    return m1, e1, m2, e2


def _gelu_exact(h):
    return 0.5 * h * (1.0 + lax.erf(h * 0.7071067811865476))


def _router_body(x_ref, gate_ref, pos0_ref, pos1_ref, w0_ref, w1_ref, be_ref):
    xt = x_ref[...]
    m1, e1, m2, e2 = _router_top2(xt, gate_ref[...])
    eidx = lax.broadcasted_iota(jnp.int32, (_T, _E), 1)
    oh0 = (eidx == e1).astype(jnp.float32)
    oh1 = (eidx == e2).astype(jnp.float32)

    TR = 512
    ti = lax.broadcasted_iota(jnp.int32, (TR, TR), 0)
    tj = lax.broadcasted_iota(jnp.int32, (TR, TR), 1)
    tri = (tj < ti).astype(jnp.float32)

    def chunked_excl_cumsum(oh):
        outs = []
        carry = jnp.zeros((1, _E), jnp.float32)
        for c in range(_T // TR):
            blk = oh[c * TR:(c + 1) * TR]
            outs.append(jnp.dot(tri, blk, preferred_element_type=jnp.float32)
                        + carry)
            carry = carry + jnp.sum(blk, axis=0, keepdims=True)
        return jnp.concatenate(outs, axis=0), carry

    ex0, c0 = chunked_excl_cumsum(oh0)
    ex1, c1 = chunked_excl_cumsum(oh1)
    rank0 = jnp.sum(oh0 * ex0, axis=1, keepdims=True)
    rank1 = jnp.sum(oh1 * (ex1 + c0), axis=1, keepdims=True)

    counts = c0 + c1
    nb = jnp.floor((counts + float(_BM - 1)) * (1.0 / _BM))
    i8 = lax.broadcasted_iota(jnp.int32, (_E, _E), 0)
    j8 = lax.broadcasted_iota(jnp.int32, (_E, _E), 1)
    tri8 = (i8 < j8).astype(jnp.float32)
    bo = jnp.dot(nb, tri8, preferred_element_type=jnp.float32)
    po = bo * float(_BM)

    pos0 = rank0 + jnp.sum(oh0 * po, axis=1, keepdims=True)
    pos1 = rank1 + jnp.sum(oh1 * po, axis=1, keepdims=True)
    pos0_ref[...] = pos0.astype(jnp.int32)
    pos1_ref[...] = pos1.astype(jnp.int32)
    w0_ref[...] = jnp.broadcast_to(m1, (_T, 16))
    w1_ref[...] = jnp.broadcast_to(m2, (_T, 16))

    jb = lax.broadcasted_iota(jnp.int32, (_NBLK, _E), 0).astype(jnp.float32)
    started = (jb >= jnp.broadcast_to(bo, (_NBLK, _E))).astype(jnp.float32)
    be = jnp.sum(started, axis=1, keepdims=True) - 1.0
    be_ref[...] = be.astype(jnp.int32)


def _router_call(x2, gate_w):
    return pl.pallas_call(
        _router_body,
        out_shape=[
            jax.ShapeDtypeStruct((_T, 1), jnp.int32),
            jax.ShapeDtypeStruct((_T, 1), jnp.int32),
            jax.ShapeDtypeStruct((_T, 16), jnp.float32),
            jax.ShapeDtypeStruct((_T, 16), jnp.float32),
            jax.ShapeDtypeStruct((_NBLK, 1), jnp.int32),
        ],
    )(x2, gate_w)


def _dispatch_sc(x2, pos0, pos1):
    mesh = plsc.VectorSubcoreMesh(core_axis_name="c", subcore_axis_name="s")

    @functools.partial(
        pl.kernel, mesh=mesh,
        out_type=jax.ShapeDtypeStruct((_NPAD, _D), jnp.float32),
        scratch_types=[
            pltpu.VMEM((_TPW,), jnp.int32),
            pltpu.VMEM((_TPW,), jnp.int32),
            pltpu.VMEM((_TPW, _D), jnp.float32),
            pltpu.SemaphoreType.DMA,
        ],
    )
    def dispatch(x_hbm, pos0_hbm, pos1_hbm, xs_hbm, idx0_v, idx1_v, rows_v,
                 sem):
        wid = lax.axis_index("s") * 2 + lax.axis_index("c")
        base = wid * _TPW
        pltpu.sync_copy(x_hbm.at[pl.ds(base, _TPW)], rows_v)
        pltpu.sync_copy(pos0_hbm.at[pl.ds(base, _TPW)], idx0_v)
        pltpu.sync_copy(pos1_hbm.at[pl.ds(base, _TPW)], idx1_v)
        c0 = pltpu.async_copy(rows_v, xs_hbm.at[idx0_v], sem)
        c1 = pltpu.async_copy(rows_v, xs_hbm.at[idx1_v], sem)
        c0.wait()
        c1.wait()

    return dispatch(x2, pos0, pos1)


def _experts_outer(be_ref, xs_hbm, w1_hbm, b1_hbm, w2_hbm, b2_hbm, ys_hbm):
    def inner(xs_blk, w1_blk, b1_blk, w2_blk, b2_blk, ys_blk):
        h = jnp.dot(xs_blk[...], w1_blk[0],
                    preferred_element_type=jnp.float32) + b1_blk[0]
        g = _gelu_exact(h)
        ys_blk[...] = lax.dot_general(
            g, w2_blk[0], (((1,), (0,)), ((), ())),
            preferred_element_type=jnp.float32) + b2_blk[0]

    lookahead = pl.Buffered(buffer_count=2, use_lookahead=True)
    pipeline = pltpu.emit_pipeline(
        inner,
        grid=(_NBLK,),
        trace_scopes=False,
        in_specs=[
            pl.BlockSpec((_BM, _D), lambda j: (j, 0),
                         pipeline_mode=pl.Buffered(buffer_count=3)),
            pl.BlockSpec((1, _D, _DFF), lambda j: (be_ref[j], 0, 0),
                         pipeline_mode=lookahead),
            pl.BlockSpec((1, 1, _DFF), lambda j: (be_ref[j], 0, 0)),
            pl.BlockSpec((1, _DFF, _D), lambda j: (be_ref[j], 0, 0),
                         pipeline_mode=lookahead),
            pl.BlockSpec((1, 1, _D), lambda j: (be_ref[j], 0, 0)),
        ],
        out_specs=[pl.BlockSpec((_BM, _D), lambda j: (j, 0))],
    )
    pipeline(xs_hbm, w1_hbm, b1_hbm, w2_hbm, b2_hbm, ys_hbm)


def _experts_call(be, xs, W1, b1, W2, b2):
    return pl.pallas_call(
        _experts_outer,
        in_specs=[
            pl.BlockSpec(memory_space=pltpu.SMEM),
            pl.BlockSpec(memory_space=pl.ANY),
            pl.BlockSpec(memory_space=pl.ANY),
            pl.BlockSpec(memory_space=pl.ANY),
            pl.BlockSpec(memory_space=pl.ANY),
            pl.BlockSpec(memory_space=pl.ANY),
        ],
        out_specs=pl.BlockSpec(memory_space=pl.ANY),
        out_shape=jax.ShapeDtypeStruct((_NPAD, _D), jnp.float32),
    )(be, xs, W1, b1.reshape(_E, 1, _DFF), W2, b2.reshape(_E, 1, _D))


def _combine_sc(ys, pos0, pos1, w0r, w1r):
    mesh = plsc.VectorSubcoreMesh(core_axis_name="c", subcore_axis_name="s")

    @functools.partial(
        pl.kernel, mesh=mesh,
        out_type=jax.ShapeDtypeStruct((_T, _D), jnp.float32),
        scratch_types=[
            pltpu.VMEM((_CHT,), jnp.int32),
            pltpu.VMEM((_CHT,), jnp.int32),
            pltpu.VMEM((_CHT, _D), jnp.float32),
            pltpu.VMEM((_CHT, _D), jnp.float32),
            pltpu.VMEM((_TPW, 16), jnp.float32),
            pltpu.VMEM((_TPW, 16), jnp.float32),
            pltpu.SemaphoreType.DMA,
        ],
    )
    def combine(ys_hbm, pos0_hbm, pos1_hbm, w0_hbm, w1_hbm, y_hbm,
                idx0_v, idx1_v, r0_v, r1_v, w0_v, w1_v, sem):
        wid = lax.axis_index("s") * 2 + lax.axis_index("c")
        base = wid * _TPW
        pltpu.sync_copy(w0_hbm.at[pl.ds(base, _TPW)], w0_v)
        pltpu.sync_copy(w1_hbm.at[pl.ds(base, _TPW)], w1_v)
        for ch in range(_TPW // _CHT):
            cbase = base + ch * _CHT
            pltpu.sync_copy(pos0_hbm.at[pl.ds(cbase, _CHT)], idx0_v)
            pltpu.sync_copy(pos1_hbm.at[pl.ds(cbase, _CHT)], idx1_v)
            g0 = pltpu.async_copy(ys_hbm.at[idx0_v], r0_v, sem)
            g1 = pltpu.async_copy(ys_hbm.at[idx1_v], r1_v, sem)
            g0.wait()
            g1.wait()

            def tok_body(t, _, ch=ch):
                wa = w0_v[ch * _CHT + t, :]
                wb = w1_v[ch * _CHT + t, :]
                for jc in range(_D // 16):
                    a = r0_v[t, pl.ds(jc * 16, 16)]
                    b = r1_v[t, pl.ds(jc * 16, 16)]
                    r0_v[t, pl.ds(jc * 16, 16)] = a * wa + b * wb
                return 0

            lax.fori_loop(0, _CHT, tok_body, 0)
            pltpu.sync_copy(r0_v, y_hbm.at[pl.ds(cbase, _CHT)])

    return combine(ys, pos0, pos1, w0r, w1r)


def kernel(x, gate_w, W1, b1, W2, b2):
    B_, T_, D_ = x.shape
    x2 = x.reshape(T_, D_)
    pos0, pos1, w0r, w1r, be = _router_call(x2, gate_w)
    pos0f = pos0.reshape(T_)
    pos1f = pos1.reshape(T_)
    xs = _dispatch_sc(x2, pos0f, pos1f)
    ys = _experts_call(be.reshape(_NBLK), xs, W1, b1, W2, b2)
    y = _combine_sc(ys, pos0f, pos1f, w0r, w1r)
    return y.reshape(B_, T_, D_)
